# baseline (speedup 1.0000x reference)
import functools
import jax
import jax.numpy as jnp
from jax import lax
from jax.experimental import pallas as pl
from jax.experimental.pallas import tpu as pltpu
from jax.experimental.pallas import tpu_sc as plsc

D_MODEL = 64
SCALE = 8.0
LANES = 16

NUM_CORES = 2
NUM_SUBCORES = 16
NUM_WORKERS = NUM_CORES * NUM_SUBCORES

BATCH = 4096 * 200
PER_WORKER = BATCH // NUM_WORKERS
CHUNK = 64
NUM_CHUNKS = PER_WORKER // CHUNK
NBUF = 4
LOOKAHEAD = 2

_mesh = plsc.VectorSubcoreMesh(core_axis_name="c", subcore_axis_name="s")


@functools.partial(
    pl.kernel,
    out_type=jax.ShapeDtypeStruct((BATCH, D_MODEL), jnp.float32),
    mesh=_mesh,
    scratch_types=[
        pltpu.VMEM((PER_WORKER,), jnp.int32),
        pltpu.VMEM((PER_WORKER,), jnp.int32),
        [pltpu.VMEM((CHUNK, 2 * D_MODEL), jnp.float32)] * NBUF,
        [pltpu.VMEM((CHUNK, D_MODEL), jnp.float32)] * NBUF,
        [pltpu.SemaphoreType.DMA] * NBUF,
        [pltpu.SemaphoreType.DMA] * NBUF,
    ],
    compiler_params=pltpu.CompilerParams(needs_layout_passes=False),
)
def _embed(phys_hbm, par_hbm, table_hbm, out_hbm, phys_v, par_v, inbuf, outbuf, gsem, ssem):
    wid = lax.axis_index("s") * NUM_CORES + lax.axis_index("c")
    base = wid * PER_WORKER
    pltpu.sync_copy(phys_hbm.at[pl.ds(base, PER_WORKER)], phys_v)
    pltpu.sync_copy(par_hbm.at[pl.ds(base, PER_WORKER)], par_v)

    def gather(g, b):
        src = table_hbm.at[phys_v.at[pl.ds(g * CHUNK, CHUNK)]]
        return pltpu.async_copy(src, inbuf[b], gsem[b])

    def scatter(g, b):
        dst = out_hbm.at[pl.ds(base + g * CHUNK, CHUNK)]
        return pltpu.make_async_copy(outbuf[b], dst, ssem[b])

    for b in range(LOOKAHEAD):
        gather(b, b)

    def outer(i, carry):
        g0 = i * NBUF
        for j in range(NBUF):
            g = g0 + j
            b = j
            b2 = (j + LOOKAHEAD) % NBUF
            gl = g + LOOKAHEAD

            @pl.when(jnp.logical_and(gl >= NBUF, gl < NUM_CHUNKS))
            def _():
                scatter(gl - NBUF, b2).wait()

            @pl.when(gl < NUM_CHUNKS)
            def _():
                gather(gl, b2)

            src = table_hbm.at[phys_v.at[pl.ds(g * CHUNK, CHUNK)]]
            pltpu.make_async_copy(src, inbuf[b], gsem[b]).wait()

            lane = jnp.arange(LANES, dtype=jnp.int32)
            for k in range(CHUNK // LANES):
                rowi = lane + (k * LANES)
                coloff = par_v[pl.ds(g * CHUNK + k * LANES, LANES)] * D_MODEL
                for c in range(D_MODEL):
                    vals = plsc.load_gather(inbuf[b], [rowi, coloff + c])
                    plsc.store_scatter(
                        outbuf[b], [rowi, jnp.full((LANES,), c, jnp.int32)], vals * SCALE
                    )

            scatter(g, b).start()
        return carry

    lax.fori_loop(0, NUM_CHUNKS // NBUF, outer, 0)

    for j in range(NBUF):
        scatter(NUM_CHUNKS - NBUF + j, j).wait()


def kernel(x, table):
    xi = x.astype(jnp.int32).reshape(-1)
    out = _embed(xi >> 1, xi & 1, jnp.reshape(table, (500000, 2 * D_MODEL)))
    return out.reshape(x.shape + (D_MODEL,))


# speedup vs baseline: 1.9405x; 1.9405x over previous
import functools
import jax
import jax.numpy as jnp
from jax import lax
from jax.experimental import pallas as pl
from jax.experimental.pallas import tpu as pltpu
from jax.experimental.pallas import tpu_sc as plsc

D_MODEL = 64
SCALE = 8.0
LANES = 16

NUM_CORES = 2
NUM_SUBCORES = 16
NUM_WORKERS = NUM_CORES * NUM_SUBCORES

BATCH = 4096 * 200
PER_WORKER = BATCH // NUM_WORKERS
CHUNK = 64
NUM_CHUNKS = PER_WORKER // CHUNK
NBUF = 4
LOOKAHEAD = 2

_mesh = plsc.VectorSubcoreMesh(core_axis_name="c", subcore_axis_name="s")


@functools.partial(
    pl.kernel,
    out_type=jax.ShapeDtypeStruct((BATCH, D_MODEL), jnp.float32),
    mesh=_mesh,
    scratch_types=[
        pltpu.VMEM((PER_WORKER,), jnp.int32),
        pltpu.VMEM((PER_WORKER,), jnp.int32),
        [pltpu.VMEM((CHUNK, 2 * D_MODEL), jnp.float32)] * NBUF,
        [pltpu.VMEM((CHUNK, D_MODEL), jnp.float32)] * NBUF,
        [pltpu.SemaphoreType.DMA] * NBUF,
        [pltpu.SemaphoreType.DMA] * NBUF,
    ],
    compiler_params=pltpu.CompilerParams(needs_layout_passes=False),
)
def _embed(phys_hbm, par_hbm, table_hbm, out_hbm, phys_v, par_v, inbuf, outbuf, gsem, ssem):
    wid = lax.axis_index("s") * NUM_CORES + lax.axis_index("c")
    base = wid * PER_WORKER
    pltpu.sync_copy(phys_hbm.at[pl.ds(base, PER_WORKER)], phys_v)
    pltpu.sync_copy(par_hbm.at[pl.ds(base, PER_WORKER)], par_v)

    def gather(g, b):
        src = table_hbm.at[phys_v.at[pl.ds(g * CHUNK, CHUNK)]]
        return pltpu.async_copy(src, inbuf[b], gsem[b])

    def scatter(g, b):
        dst = out_hbm.at[pl.ds(base + g * CHUNK, CHUNK)]
        return pltpu.make_async_copy(outbuf[b], dst, ssem[b])

    for b in range(LOOKAHEAD):
        gather(b, b)

    def outer(i, carry):
        g0 = i * NBUF
        for j in range(NBUF):
            g = g0 + j
            b = j
            b2 = (j + LOOKAHEAD) % NBUF
            gl = g + LOOKAHEAD

            @pl.when(jnp.logical_and(gl >= NBUF, gl < NUM_CHUNKS))
            def _():
                scatter(gl - NBUF, b2).wait()

            @pl.when(gl < NUM_CHUNKS)
            def _():
                gather(gl, b2)

            src = table_hbm.at[phys_v.at[pl.ds(g * CHUNK, CHUNK)]]
            pltpu.make_async_copy(src, inbuf[b], gsem[b]).wait()

            for k in range(CHUNK // LANES):
                coloff = par_v[pl.ds(g * CHUNK + k * LANES, LANES)] * D_MODEL
                for m in range(LANES):
                    rr = k * LANES + m
                    off = coloff[m]
                    for jj in range(D_MODEL // LANES):
                        outbuf[b][rr, pl.ds(jj * LANES, LANES)] = (
                            inbuf[b][rr, pl.ds(off + jj * LANES, LANES)] * SCALE
                        )

            scatter(g, b).start()
        return carry

    lax.fori_loop(0, NUM_CHUNKS // NBUF, outer, 0)

    for j in range(NBUF):
        scatter(NUM_CHUNKS - NBUF + j, j).wait()


def kernel(x, table):
    xi = x.astype(jnp.int32).reshape(-1)
    out = _embed(xi >> 1, xi & 1, jnp.reshape(table, (500000, 2 * D_MODEL)))
    return out.reshape(x.shape + (D_MODEL,))


# speedup vs baseline: 2.3872x; 1.2302x over previous
import functools
import jax
import jax.numpy as jnp
from jax import lax
from jax.experimental import pallas as pl
from jax.experimental.pallas import tpu as pltpu
from jax.experimental.pallas import tpu_sc as plsc

D_MODEL = 64
SCALE = 8.0
LANES = 16

NUM_CORES = 2
NUM_SUBCORES = 16
NUM_WORKERS = NUM_CORES * NUM_SUBCORES

BATCH = 4096 * 200
PER_WORKER = BATCH // NUM_WORKERS
CHUNK = 64
NUM_CHUNKS = PER_WORKER // CHUNK
NBUF = 4
LOOKAHEAD = 2

_mesh = plsc.VectorSubcoreMesh(core_axis_name="c", subcore_axis_name="s")


@functools.partial(
    pl.kernel,
    out_type=jax.ShapeDtypeStruct((BATCH, D_MODEL), jnp.float32),
    mesh=_mesh,
    scratch_types=[
        pltpu.VMEM((PER_WORKER,), jnp.int32),
        pltpu.VMEM((PER_WORKER,), jnp.int32),
        [pltpu.VMEM((CHUNK, 2 * D_MODEL), jnp.float32)] * NBUF,
        [pltpu.VMEM((CHUNK, D_MODEL), jnp.float32)] * NBUF,
        [pltpu.SemaphoreType.DMA] * NBUF,
        [pltpu.SemaphoreType.DMA] * NBUF,
    ],
    compiler_params=pltpu.CompilerParams(needs_layout_passes=False),
)
def _embed(phys_hbm, par_hbm, table_hbm, out_hbm, phys_v, par_v, inbuf, outbuf, gsem, ssem):
    wid = lax.axis_index("s") * NUM_CORES + lax.axis_index("c")
    base = wid * PER_WORKER
    pltpu.sync_copy(phys_hbm.at[pl.ds(base, PER_WORKER)], phys_v)
    pltpu.sync_copy(par_hbm.at[pl.ds(base, PER_WORKER)], par_v)

    def gather(g, b):
        src = table_hbm.at[phys_v.at[pl.ds(g * CHUNK, CHUNK)]]
        return pltpu.async_copy(src, inbuf[b], gsem[b])

    def scatter(g, b):
        dst = out_hbm.at[pl.ds(base + g * CHUNK, CHUNK)]
        return pltpu.make_async_copy(outbuf[b], dst, ssem[b])

    for b in range(LOOKAHEAD):
        gather(b, b)

    def outer(i, carry):
        g0 = i * NBUF
        for j in range(NBUF):
            g = g0 + j
            b = j
            b2 = (j + LOOKAHEAD) % NBUF
            gl = g + LOOKAHEAD

            @pl.when(jnp.logical_and(gl >= NBUF, gl < NUM_CHUNKS))
            def _():
                scatter(gl - NBUF, b2).wait()

            @pl.when(gl < NUM_CHUNKS)
            def _():
                gather(gl, b2)

            src = table_hbm.at[phys_v.at[pl.ds(g * CHUNK, CHUNK)]]
            pltpu.make_async_copy(src, inbuf[b], gsem[b]).wait()

            for k in range(CHUNK // LANES):
                pvec = par_v[pl.ds(g * CHUNK + k * LANES, LANES)]
                for m in range(LANES):
                    rr = k * LANES + m
                    pm = pvec[jnp.full((LANES,), m, jnp.int32)]
                    odd = pm != 0
                    for jj in range(D_MODEL // LANES):
                        lo = inbuf[b][rr, pl.ds(jj * LANES, LANES)]
                        hi = inbuf[b][rr, pl.ds(D_MODEL + jj * LANES, LANES)]
                        outbuf[b][rr, pl.ds(jj * LANES, LANES)] = (
                            jnp.where(odd, hi, lo) * SCALE
                        )

            scatter(g, b).start()
        return carry

    lax.fori_loop(0, NUM_CHUNKS // NBUF, outer, 0)

    for j in range(NBUF):
        scatter(NUM_CHUNKS - NBUF + j, j).wait()


def kernel(x, table):
    xi = x.astype(jnp.int32).reshape(-1)
    out = _embed(xi >> 1, xi & 1, jnp.reshape(table, (500000, 2 * D_MODEL)))
    return out.reshape(x.shape + (D_MODEL,))
